# baseline (speedup 1.0000x reference)
import jax
import jax.numpy as jnp
from jax import lax
from jax.experimental import pallas as pl
from jax.experimental.pallas import tpu as pltpu
from jax.experimental.pallas import tpu_sc as plsc

_NUM_USERS = 1000000
_CODE_LEN = 8
_EMB_DIM = 64
_BATCH = 4096

_LANES = 16
_NW = 32
_UPW = _BATCH // _NW
_CPW = _UPW * _CODE_LEN
_SLOT = 2 * _CPW + 32
_WUS = 8
_NWV = _UPW // _WUS
_WCD = _WUS * _CODE_LEN
_GSZ = 128
_NG = 2 * _CPW // _GSZ
_UPG = _GSZ // _CODE_LEN
_NCH = _EMB_DIM // _LANES


def _codes_body(uid_hbm, u2r_hbm, u2s_hbm, codes_hbm,
                uid_v, ucol_v, blk_r0, blk_s0, blk_r1, blk_s1, cflat,
                sem_u0, sem_u1):
    wid = lax.axis_index("s") * 2 + lax.axis_index("c")
    base = wid * _UPW

    pltpu.sync_copy(uid_hbm.at[pl.ds(base, _UPW)], uid_v.at[pl.ds(0, _UPW)])

    def fetch_wave(w, blk_r, blk_s, sem):
        uv = uid_v[pl.ds(w * _WUS, _LANES)]
        ucol_v[pl.ds(w * _WUS, _LANES)] = uv & 127
        for i in range(_WUS):
            s = pl.multiple_of(uv[i] & -128, 128)
            pltpu.async_copy(u2r_hbm.at[:, pl.ds(s, 128)], blk_r.at[i], sem)
            pltpu.async_copy(u2s_hbm.at[:, pl.ds(s, 128)], blk_s.at[i], sem)

    def drain_wave(blk_r, blk_s, sem):
        @pl.loop(0, _WUS)
        def _drain(i):
            pltpu.make_async_copy(
                u2r_hbm.at[:, pl.ds(0, 128)], blk_r.at[i], sem).wait()
            pltpu.make_async_copy(
                u2r_hbm.at[:, pl.ds(0, 128)], blk_s.at[i], sem).wait()

    def extract_wave(w, blk_r, blk_s):
        @pl.loop(0, _WCD // _LANES)
        def _ext(i):
            k = lax.iota(jnp.int32, _LANES) + w * _WCD + i * _LANES
            u = k >> 3
            uu = u - w * _WUS
            l = k & 7
            col = plsc.load_gather(ucol_v, [u])
            cflat[pl.ds(w * _WCD + i * _LANES, _LANES)] = (
                plsc.load_gather(blk_r, [uu, l, col]))
            cflat[pl.ds(_CPW + w * _WCD + i * _LANES, _LANES)] = (
                plsc.load_gather(blk_s, [uu, l, col]))

    fetch_wave(0, blk_r0, blk_s0, sem_u0)

    @pl.loop(0, _NWV, step=2)
    def _waves(w0):
        for b, (blk_r, blk_s, sem_u, nblk_r, nblk_s, nsem_u) in enumerate((
                (blk_r0, blk_s0, sem_u0, blk_r1, blk_s1, sem_u1),
                (blk_r1, blk_s1, sem_u1, blk_r0, blk_s0, sem_u0))):
            w = w0 + b

            @pl.when(w + 1 < _NWV)
            def _():
                fetch_wave(w + 1, nblk_r, nblk_s, nsem_u)

            drain_wave(blk_r, blk_s, sem_u)
            extract_wave(w, blk_r, blk_s)

    pltpu.sync_copy(cflat, codes_hbm.at[pl.ds(wid * _SLOT, _SLOT)])


def _pool_body(codes_hbm, emb_hbm, out_hbm,
               cflat, buf0, buf1, outbuf, emb0_v, sem0, sem1):
    wid = lax.axis_index("s") * 2 + lax.axis_index("c")

    pltpu.sync_copy(codes_hbm.at[pl.ds(wid * _SLOT, _SLOT)], cflat)
    pltpu.sync_copy(emb_hbm.at[0], emb0_v)
    e0 = [emb0_v[pl.ds(c * _LANES, _LANES)] for c in range(_NCH)]

    def start_gather(g, buf, sem):
        pltpu.async_copy(emb_hbm.at[cflat.at[pl.ds(g * _GSZ, _GSZ)]], buf, sem)

    def accumulate(g, buf):
        t = g >> 3
        g8 = g & 7

        @pl.loop(0, _UPG)
        def _users(u):
            row0 = u * _CODE_LEN
            acc = [buf[row0, pl.ds(c * _LANES, _LANES)]
                   for c in range(_NCH)]
            for l in range(1, _CODE_LEN):
                for c in range(_NCH):
                    acc[c] = acc[c] + buf[row0 + l, pl.ds(c * _LANES, _LANES)]
            cb = g * _GSZ + row0
            cv = cflat[pl.ds(cb, _LANES)]
            zmask = (cv == 0) & (lax.iota(jnp.int32, _LANES) < _CODE_LEN)
            zf = jnp.sum(jnp.where(zmask, 1.0, 0.0).astype(jnp.float32))
            urow = g8 * _UPG + u
            cbase = t * _EMB_DIM
            for c in range(_NCH):
                outbuf[urow, pl.ds(cbase + c * _LANES, _LANES)] = (
                    acc[c] - zf * e0[c])

    start_gather(0, buf0, sem0)

    @pl.loop(0, _NG, step=2)
    def _groups(g0):
        for b, (buf, sem, nbuf, nsem) in enumerate(
                ((buf0, sem0, buf1, sem1), (buf1, sem1, buf0, sem0))):
            g = g0 + b

            @pl.when(g + 1 < _NG)
            def _():
                start_gather(g + 1, nbuf, nsem)

            pltpu.make_async_copy(
                emb_hbm.at[pl.ds(0, _GSZ)], buf, sem).wait()
            accumulate(g, buf)

    pltpu.sync_copy(outbuf, out_hbm.at[wid])


def kernel(user_ids, user2rec_code, user2src_code, code_embedding):
    mesh = plsc.VectorSubcoreMesh(core_axis_name="c", subcore_axis_name="s")
    codes = pl.kernel(
        _codes_body,
        out_type=jax.ShapeDtypeStruct((_NW * _SLOT,), jnp.int32),
        mesh=mesh,
        compiler_params=pltpu.CompilerParams(
            needs_layout_passes=False, disable_bounds_checks=True),
        scratch_types=[
            pltpu.VMEM((_UPW + _LANES,), jnp.int32),
            pltpu.VMEM((_UPW + _LANES,), jnp.int32),
            pltpu.VMEM((_WUS, _CODE_LEN, 128), jnp.int32),
            pltpu.VMEM((_WUS, _CODE_LEN, 128), jnp.int32),
            pltpu.VMEM((_WUS, _CODE_LEN, 128), jnp.int32),
            pltpu.VMEM((_WUS, _CODE_LEN, 128), jnp.int32),
            pltpu.VMEM((_SLOT,), jnp.int32),
            pltpu.SemaphoreType.DMA,
            pltpu.SemaphoreType.DMA,
        ],
    )(user_ids, user2rec_code.T, user2src_code.T)

    out = pl.kernel(
        _pool_body,
        out_type=jax.ShapeDtypeStruct((_NW, _UPW, 2 * _EMB_DIM), jnp.float32),
        mesh=mesh,
        compiler_params=pltpu.CompilerParams(
            needs_layout_passes=False, use_tc_tiling_on_sc=False,
            disable_bounds_checks=True),
        scratch_types=[
            pltpu.VMEM((_SLOT,), jnp.int32),
            pltpu.VMEM((_GSZ, _EMB_DIM), jnp.float32),
            pltpu.VMEM((_GSZ, _EMB_DIM), jnp.float32),
            pltpu.VMEM((_UPW, 2 * _EMB_DIM), jnp.float32),
            pltpu.VMEM((_EMB_DIM,), jnp.float32),
            pltpu.SemaphoreType.DMA,
            pltpu.SemaphoreType.DMA,
        ],
    )(codes, code_embedding)
    return out.reshape(_BATCH, 2 * _EMB_DIM)


# speedup vs baseline: 1.2595x; 1.2595x over previous
import jax
import jax.numpy as jnp
from jax import lax
from jax.experimental import pallas as pl
from jax.experimental.pallas import tpu as pltpu
from jax.experimental.pallas import tpu_sc as plsc

_NUM_USERS = 1000000
_CODE_LEN = 8
_EMB_DIM = 64
_BATCH = 4096

_LANES = 16
_NW = 32
_UPW = _BATCH // _NW
_CPW = _UPW * _CODE_LEN
_SLOT = 2 * _CPW + 32
_WUS = 8
_NWV = _UPW // _WUS
_WCD = _WUS * _CODE_LEN
_GSZ = 128
_NG = 2 * _CPW // _GSZ
_UPG = _GSZ // _CODE_LEN
_NCH = _EMB_DIM // _LANES


def _codes_body(uid_hbm, u2r_hbm, u2s_hbm, codes_hbm,
                uid_v, ucol_v, blk_r0, blk_s0, blk_r1, blk_s1, cflat,
                sem_u0, sem_u1):
    wid = lax.axis_index("s") * 2 + lax.axis_index("c")
    base = wid * _UPW

    pltpu.sync_copy(uid_hbm.at[pl.ds(base, _UPW)], uid_v.at[pl.ds(0, _UPW)])

    def fetch_wave(w, blk_r, blk_s, sem):
        uv = uid_v[pl.ds(w * _WUS, _LANES)]
        ucol_v[pl.ds(w * _WUS, _LANES)] = uv & 127
        for i in range(_WUS):
            s = pl.multiple_of(uv[i] & -128, 128)
            pltpu.async_copy(u2r_hbm.at[:, pl.ds(s, 128)], blk_r.at[i], sem)
            pltpu.async_copy(u2s_hbm.at[:, pl.ds(s, 128)], blk_s.at[i], sem)

    def drain_wave(blk_r, blk_s, sem):
        @pl.loop(0, _WUS)
        def _drain(i):
            pltpu.make_async_copy(
                u2r_hbm.at[:, pl.ds(0, 128)], blk_r.at[i], sem).wait()
            pltpu.make_async_copy(
                u2r_hbm.at[:, pl.ds(0, 128)], blk_s.at[i], sem).wait()

    def extract_wave(w, blk_r, blk_s):
        @pl.loop(0, _WCD // _LANES)
        def _ext(i):
            k = lax.iota(jnp.int32, _LANES) + w * _WCD + i * _LANES
            u = k >> 3
            uu = u - w * _WUS
            l = k & 7
            col = plsc.load_gather(ucol_v, [u])
            cflat[pl.ds(w * _WCD + i * _LANES, _LANES)] = (
                plsc.load_gather(blk_r, [uu, l, col]))
            cflat[pl.ds(_CPW + w * _WCD + i * _LANES, _LANES)] = (
                plsc.load_gather(blk_s, [uu, l, col]))

    fetch_wave(0, blk_r0, blk_s0, sem_u0)

    @pl.loop(0, _NWV, step=2)
    def _waves(w0):
        for b, (blk_r, blk_s, sem_u, nblk_r, nblk_s, nsem_u) in enumerate((
                (blk_r0, blk_s0, sem_u0, blk_r1, blk_s1, sem_u1),
                (blk_r1, blk_s1, sem_u1, blk_r0, blk_s0, sem_u0))):
            w = w0 + b

            @pl.when(w + 1 < _NWV)
            def _():
                fetch_wave(w + 1, nblk_r, nblk_s, nsem_u)

            drain_wave(blk_r, blk_s, sem_u)
            extract_wave(w, blk_r, blk_s)

    pltpu.sync_copy(cflat, codes_hbm.at[pl.ds(wid * _SLOT, _SLOT)])


def _pool_body(codes_hbm, emb_hbm, out_hbm,
               cflat, buf0, buf1, outbuf, emb0_v, sem0, sem1):
    wid = lax.axis_index("s") * 2 + lax.axis_index("c")

    pltpu.sync_copy(codes_hbm.at[pl.ds(wid * _SLOT, _SLOT)], cflat)
    pltpu.sync_copy(emb_hbm.at[0], emb0_v)
    e0 = [emb0_v[pl.ds(c * _LANES, _LANES)] for c in range(_NCH)]

    def start_gather(g, buf, sem):
        @pl.loop(0, _GSZ // _LANES)
        def _enq(j):
            cv = cflat[pl.ds(g * _GSZ + j * _LANES, _LANES)]
            for t in range(_LANES):
                pltpu.async_copy(emb_hbm.at[cv[t]], buf.at[j * _LANES + t],
                                 sem)

    def accumulate(g, buf):
        t = g >> 3
        g8 = g & 7

        @pl.loop(0, _UPG)
        def _users(u):
            row0 = u * _CODE_LEN
            acc = [buf[row0, pl.ds(c * _LANES, _LANES)]
                   for c in range(_NCH)]
            for l in range(1, _CODE_LEN):
                for c in range(_NCH):
                    acc[c] = acc[c] + buf[row0 + l, pl.ds(c * _LANES, _LANES)]
            cb = g * _GSZ + row0
            cv = cflat[pl.ds(cb, _LANES)]
            zmask = (cv == 0) & (lax.iota(jnp.int32, _LANES) < _CODE_LEN)
            zf = jnp.sum(jnp.where(zmask, 1.0, 0.0).astype(jnp.float32))
            urow = g8 * _UPG + u
            cbase = t * _EMB_DIM
            for c in range(_NCH):
                outbuf[urow, pl.ds(cbase + c * _LANES, _LANES)] = (
                    acc[c] - zf * e0[c])

    start_gather(0, buf0, sem0)

    @pl.loop(0, _NG, step=2)
    def _groups(g0):
        for b, (buf, sem, nbuf, nsem) in enumerate(
                ((buf0, sem0, buf1, sem1), (buf1, sem1, buf0, sem0))):
            g = g0 + b

            @pl.when(g + 1 < _NG)
            def _():
                start_gather(g + 1, nbuf, nsem)

            pltpu.make_async_copy(
                emb_hbm.at[pl.ds(0, _GSZ)], buf, sem).wait()
            accumulate(g, buf)

    pltpu.sync_copy(outbuf, out_hbm.at[wid])


def kernel(user_ids, user2rec_code, user2src_code, code_embedding):
    mesh = plsc.VectorSubcoreMesh(core_axis_name="c", subcore_axis_name="s")
    codes = pl.kernel(
        _codes_body,
        out_type=jax.ShapeDtypeStruct((_NW * _SLOT,), jnp.int32),
        mesh=mesh,
        compiler_params=pltpu.CompilerParams(
            needs_layout_passes=False, disable_bounds_checks=True),
        scratch_types=[
            pltpu.VMEM((_UPW + _LANES,), jnp.int32),
            pltpu.VMEM((_UPW + _LANES,), jnp.int32),
            pltpu.VMEM((_WUS, _CODE_LEN, 128), jnp.int32),
            pltpu.VMEM((_WUS, _CODE_LEN, 128), jnp.int32),
            pltpu.VMEM((_WUS, _CODE_LEN, 128), jnp.int32),
            pltpu.VMEM((_WUS, _CODE_LEN, 128), jnp.int32),
            pltpu.VMEM((_SLOT,), jnp.int32),
            pltpu.SemaphoreType.DMA,
            pltpu.SemaphoreType.DMA,
        ],
    )(user_ids, user2rec_code.T, user2src_code.T)

    out = pl.kernel(
        _pool_body,
        out_type=jax.ShapeDtypeStruct((_NW, _UPW, 2 * _EMB_DIM), jnp.float32),
        mesh=mesh,
        compiler_params=pltpu.CompilerParams(
            needs_layout_passes=False, disable_bounds_checks=True),
        scratch_types=[
            pltpu.VMEM((_SLOT,), jnp.int32),
            pltpu.VMEM((_GSZ, _EMB_DIM), jnp.float32),
            pltpu.VMEM((_GSZ, _EMB_DIM), jnp.float32),
            pltpu.VMEM((_UPW, 2 * _EMB_DIM), jnp.float32),
            pltpu.VMEM((_EMB_DIM,), jnp.float32),
            pltpu.SemaphoreType.DMA,
            pltpu.SemaphoreType.DMA,
        ],
    )(codes, code_embedding)
    return out.reshape(_BATCH, 2 * _EMB_DIM)
